# baseline (speedup 1.0000x reference)
import functools

import jax
import jax.numpy as jnp
from jax import lax
from jax.experimental import pallas as pl
from jax.experimental.pallas import tpu as pltpu
from jax.experimental.pallas import tpu_sc as plsc

_NC, _NS, _L = 2, 16, 16
_NW = _NC * _NS
_CH = 3
_NST = 8


@functools.lru_cache(maxsize=None)
def _make_extract(B, D, V):
    tcn = (V + 127) // 128
    mesh = plsc.VectorSubcoreMesh(
        core_axis_name="c", subcore_axis_name="s",
        num_cores=_NC, num_subcores=_NS,
    )

    @functools.partial(
        pl.kernel,
        out_type=(jax.ShapeDtypeStruct((B * D,), jnp.float32),
                  jax.ShapeDtypeStruct((B * D,), jnp.float32)),
        mesh=mesh,
        scratch_types=[
            pltpu.VMEM((2048,), jnp.int32),
            pltpu.VMEM((B + _L,), jnp.int32),
            pltpu.VMEM((D, _CH * 128), jnp.float32),
            pltpu.VMEM((D, _CH * 128), jnp.float32),
            pltpu.VMEM((D, _CH * 128), jnp.float32),
            pltpu.VMEM((D, _CH * 128), jnp.float32),
            pltpu.VMEM((_L,), jnp.int32),
            pltpu.VMEM((_NST * D,), jnp.float32),
            pltpu.SMEM((1,), jnp.int32),
            pltpu.SMEM((1,), jnp.int32),
            pltpu.SemaphoreType.DMA,
            pltpu.SemaphoreType.DMA,
            pltpu.SemaphoreType.DMA,
            pltpu.SemaphoreType.DMA,
            pltpu.SemaphoreType.DMA,
        ],
        compiler_params=pltpu.CompilerParams(needs_layout_passes=False),
    )
    def k(uid_hbm, mid_hbm, utt_hbm, mtt_hbm, uscr_hbm, mscr_hbm,
          ids_v, lpk_v, bufa_v, bufb_v, bufc_v, bufd_v, hpk_v, stage_v,
          cnt_s, iss_s, sem_a, sem_b, sem_c, sem_d, sem_w):
        wid = lax.axis_index("s") * _NC + lax.axis_index("c")
        lo = (wid * tcn) // _NW
        hi = ((wid + 1) * tcn) // _NW
        lane = lax.broadcasted_iota(jnp.int32, (_L,), 0)
        dchunks = [lane + c * _L for c in range(D // _L)]
        iss_s[0] = 0
        bufs = [(bufa_v, sem_a), (bufb_v, sem_b), (bufc_v, sem_c),
                (bufd_v, sem_d)]

        def phase(id_hbm, tab_hbm, scr_hbm):
            nch = (hi - lo + _CH - 1) // _CH

            def fire(cc, buf, sem):
                tc0 = lo + cc * _CH
                cb = pl.multiple_of(tc0 << 7, 128)
                return pltpu.async_copy(tab_hbm.at[:, pl.ds(cb, _CH * 128)],
                                        buf, sem)

            fire(0, bufa_v, sem_a)

            @pl.when(nch > 1)
            def _():
                fire(1, bufb_v, sem_b)

            @pl.when(nch > 2)
            def _():
                fire(2, bufc_v, sem_c)

            cnt_s[0] = 0

            def slice_scan(sl, carry):
                pltpu.sync_copy(id_hbm.at[pl.ds(sl * 2048, 2048)], ids_v)

                def scan(g, carry2):
                    idv = ids_v[pl.ds(g * _L, _L)]
                    tcv = idv >> 7
                    m = (tcv >= lo) & (tcv < hi)
                    cnt = cnt_s[0]
                    pos = lane + (sl * 2048 + g * _L)
                    packv = (((tcv - lo) << 21) | ((idv & 127) << 14) | pos)
                    plsc.store_compressed(lpk_v.at[pl.ds(cnt, _L)], packv,
                                          mask=m)
                    n = plsc.all_reduce_population_count(m)
                    cnt_s[0] = cnt + n[0]
                    return carry2

                lax.fori_loop(0, 2048 // _L, scan, 0)
                return carry

            lax.fori_loop(0, B // 2048, slice_scan, 0)
            cnt = cnt_s[0]
            ngr = (cnt + _L - 1) // _L

            def process(cc, buf_v):
                rel0 = cc * _CH

                def group(gg, carry2):
                    lv = lpk_v[pl.ds(gg * _L, _L)]
                    rel = (lv >> 21) - rel0
                    m2 = (rel >= 0) & (rel < _CH) & ((gg * _L + lane) < cnt)
                    nh = plsc.all_reduce_population_count(m2)[0]

                    @pl.when(nh > 0)
                    def _():
                        plsc.store_compressed(hpk_v.at[pl.ds(0, _L)], lv,
                                              mask=m2)

                        def hit(h, carry3):
                            hsplat = jnp.full((_L,), h, jnp.int32)
                            pk = plsc.load_gather(hpk_v, [hsplat])
                            colsp = (((pk >> 21) - rel0) << 7) | ((pk >> 14)
                                                                  & 127)
                            pos = (pk & 16383)[0]
                            iss = iss_s[0]
                            soff = pl.multiple_of((iss % _NST) * D, 8)
                            for c in range(D // _L):
                                v = plsc.load_gather(buf_v,
                                                     [dchunks[c], colsp])
                                stage_v[pl.ds(soff + c * _L, _L)] = v
                            @pl.when(iss >= _NST)
                            def _():
                                pltpu.make_async_copy(
                                    scr_hbm.at[pl.ds(0, D)],
                                    stage_v.at[pl.ds(0, D)], sem_w).wait()
                            pltpu.async_copy(
                                stage_v.at[pl.ds(soff, D)],
                                scr_hbm.at[pl.ds(pos * D, D)], sem_w)
                            iss_s[0] = iss + 1
                            return carry3

                        lax.fori_loop(0, nh, hit, 0)

                    return carry2

                lax.fori_loop(0, ngr, group, 0)

            def waitc(buf, sem):
                pltpu.make_async_copy(
                    tab_hbm.at[:, pl.ds(0, _CH * 128)], buf, sem).wait()

            nit = (nch + 3) // 4

            def it_body(it, carry):
                for q in range(4):
                    cc = it * 4 + q
                    buf, sem = bufs[q]
                    nbuf, nsem = bufs[(q + 3) % 4]

                    @pl.when(cc < nch)
                    def _():
                        waitc(buf, sem)

                        @pl.when(cc + 3 < nch)
                        def _():
                            fire(cc + 3, nbuf, nsem)

                        process(cc, buf)

                return carry

            lax.fori_loop(0, nit, it_body, 0)

        phase(uid_hbm, utt_hbm, uscr_hbm)
        phase(mid_hbm, mtt_hbm, mscr_hbm)

        rem = jnp.minimum(iss_s[0], _NST)

        def drain(i, carry):
            pltpu.make_async_copy(uscr_hbm.at[pl.ds(0, D)],
                                  stage_v.at[pl.ds(0, D)], sem_w).wait()
            return carry

        lax.fori_loop(0, rem, drain, 0)

    return k


@functools.lru_cache(maxsize=None)
def _make_dot(B, D):
    b_per_w = B // _NW
    n_groups = b_per_w // _L
    mesh = plsc.VectorSubcoreMesh(
        core_axis_name="c", subcore_axis_name="s",
        num_cores=_NC, num_subcores=_NS,
    )

    @functools.partial(
        pl.kernel,
        out_type=jax.ShapeDtypeStruct((B,), jnp.float32),
        mesh=mesh,
        scratch_types=[
            pltpu.VMEM((b_per_w * D,), jnp.float32),
            pltpu.VMEM((b_per_w * D,), jnp.float32),
            pltpu.VMEM((b_per_w,), jnp.float32),
            pltpu.SemaphoreType.DMA,
            pltpu.SemaphoreType.DMA,
        ],
        compiler_params=pltpu.CompilerParams(needs_layout_passes=False),
    )
    def k(uscr_hbm, mscr_hbm, out_hbm, ubuf_v, mbuf_v, res_v, sem_u, sem_m):
        wid = lax.axis_index("s") * _NC + lax.axis_index("c")
        base = wid * b_per_w
        cu = pltpu.async_copy(uscr_hbm.at[pl.ds(base * D, b_per_w * D)],
                              ubuf_v, sem_u)
        cm = pltpu.async_copy(mscr_hbm.at[pl.ds(base * D, b_per_w * D)],
                              mbuf_v, sem_m)
        cu.wait()
        cm.wait()
        lane = lax.broadcasted_iota(jnp.int32, (_L,), 0)

        def body(g, carry):
            acc = jnp.zeros((_L,), jnp.float32)
            for j in range(_L):
                r = (g * _L + j) * D
                p = (ubuf_v[pl.ds(r, _L)] * mbuf_v[pl.ds(r, _L)])
                for c in range(1, D // _L):
                    p = p + (ubuf_v[pl.ds(r + c * _L, _L)]
                             * mbuf_v[pl.ds(r + c * _L, _L)])
                s = jnp.sum(p)
                acc = jnp.where(lane == j, s, acc)
            res_v[pl.ds(g * _L, _L)] = acc
            return carry

        lax.fori_loop(0, n_groups, body, 0)
        pltpu.sync_copy(res_v, out_hbm.at[pl.ds(base, b_per_w)])

    return k


def kernel(user_id, movie_id, user_table, movie_table):
    B = user_id.shape[0]
    V, D = user_table.shape
    uscr, mscr = _make_extract(B, D, V)(user_id, movie_id,
                                        user_table.T, movie_table.T)
    out = _make_dot(B, D)(uscr, mscr)
    return out.reshape(B, 1)


# speedup vs baseline: 1.1376x; 1.1376x over previous
import functools

import jax
import jax.numpy as jnp
from jax import lax
from jax.experimental import pallas as pl
from jax.experimental.pallas import tpu as pltpu
from jax.experimental.pallas import tpu_sc as plsc

_NC, _NS, _L = 2, 16, 16
_NW = _NC * _NS
_CH = 4
_NST = 8


@functools.lru_cache(maxsize=None)
def _make_extract(B, D, V):
    tcn = (V + 127) // 128
    mesh = plsc.VectorSubcoreMesh(
        core_axis_name="c", subcore_axis_name="s",
        num_cores=_NC, num_subcores=_NS,
    )

    @functools.partial(
        pl.kernel,
        out_type=(jax.ShapeDtypeStruct((B * D,), jnp.float32),
                  jax.ShapeDtypeStruct((B * D,), jnp.float32)),
        mesh=mesh,
        scratch_types=[
            pltpu.VMEM((2048,), jnp.int32),
            pltpu.VMEM((B + _L,), jnp.int32),
            pltpu.VMEM((D, _CH * 128), jnp.float32),
            pltpu.VMEM((D, _CH * 128), jnp.float32),
            pltpu.VMEM((D, _CH * 128), jnp.float32),
            pltpu.VMEM((_L,), jnp.int32),
            pltpu.VMEM((_NST * D,), jnp.float32),
            pltpu.SMEM((1,), jnp.int32),
            pltpu.SMEM((1,), jnp.int32),
            pltpu.SemaphoreType.DMA,
            pltpu.SemaphoreType.DMA,
            pltpu.SemaphoreType.DMA,
            pltpu.SemaphoreType.DMA,
        ],
        compiler_params=pltpu.CompilerParams(needs_layout_passes=False),
    )
    def k(uid_hbm, mid_hbm, utt_hbm, mtt_hbm, uscr_hbm, mscr_hbm,
          ids_v, lpk_v, bufa_v, bufb_v, bufc_v, hpk_v, stage_v,
          cnt_s, iss_s, sem_a, sem_b, sem_c, sem_w):
        wid = lax.axis_index("s") * _NC + lax.axis_index("c")
        lo = (wid * tcn) // _NW
        hi = ((wid + 1) * tcn) // _NW
        lane = lax.broadcasted_iota(jnp.int32, (_L,), 0)
        dchunks = [lane + c * _L for c in range(D // _L)]
        iss_s[0] = 0
        bufs = [(bufa_v, sem_a), (bufb_v, sem_b), (bufc_v, sem_c)]

        def phase(id_hbm, tab_hbm, scr_hbm):
            nch = (hi - lo + _CH - 1) // _CH

            def fire(cc, buf, sem):
                tc0 = lo + cc * _CH
                cb = pl.multiple_of(tc0 << 7, 128)
                return pltpu.async_copy(tab_hbm.at[:, pl.ds(cb, _CH * 128)],
                                        buf, sem)

            fire(0, bufa_v, sem_a)

            @pl.when(nch > 1)
            def _():
                fire(1, bufb_v, sem_b)

            cnt_s[0] = 0

            def slice_scan(sl, carry):
                pltpu.sync_copy(id_hbm.at[pl.ds(sl * 2048, 2048)], ids_v)

                def scan(g, carry2):
                    idv = ids_v[pl.ds(g * _L, _L)]
                    tcv = idv >> 7
                    m = (tcv >= lo) & (tcv < hi)
                    cnt = cnt_s[0]
                    pos = lane + (sl * 2048 + g * _L)
                    packv = (((tcv - lo) << 21) | ((idv & 127) << 14) | pos)
                    plsc.store_compressed(lpk_v.at[pl.ds(cnt, _L)], packv,
                                          mask=m)
                    n = plsc.all_reduce_population_count(m)
                    cnt_s[0] = cnt + n[0]
                    return carry2

                lax.fori_loop(0, 2048 // _L, scan, 0)
                return carry

            lax.fori_loop(0, B // 2048, slice_scan, 0)
            cnt = cnt_s[0]
            ngr = (cnt + _L - 1) // _L

            def process(cc, buf_v):
                rel0 = cc * _CH

                def group(gg, carry2):
                    lv = lpk_v[pl.ds(gg * _L, _L)]
                    rel = (lv >> 21) - rel0
                    m2 = (rel >= 0) & (rel < _CH) & ((gg * _L + lane) < cnt)
                    nh = plsc.all_reduce_population_count(m2)[0]

                    @pl.when(nh > 0)
                    def _():
                        plsc.store_compressed(hpk_v.at[pl.ds(0, _L)], lv,
                                              mask=m2)

                        def hit(h, carry3):
                            hsplat = jnp.full((_L,), h, jnp.int32)
                            pk = plsc.load_gather(hpk_v, [hsplat])
                            colsp = (((pk >> 21) - rel0) << 7) | ((pk >> 14)
                                                                  & 127)
                            pos = (pk & 16383)[0]
                            iss = iss_s[0]
                            soff = pl.multiple_of((iss % _NST) * D, 8)
                            for c in range(D // _L):
                                v = plsc.load_gather(buf_v,
                                                     [dchunks[c], colsp])
                                stage_v[pl.ds(soff + c * _L, _L)] = v
                            @pl.when(iss >= _NST)
                            def _():
                                pltpu.make_async_copy(
                                    scr_hbm.at[pl.ds(0, D)],
                                    stage_v.at[pl.ds(0, D)], sem_w).wait()
                            pltpu.async_copy(
                                stage_v.at[pl.ds(soff, D)],
                                scr_hbm.at[pl.ds(pos * D, D)], sem_w)
                            iss_s[0] = iss + 1
                            return carry3

                        lax.fori_loop(0, nh, hit, 0)

                    return carry2

                lax.fori_loop(0, ngr, group, 0)

            def waitc(buf, sem):
                pltpu.make_async_copy(
                    tab_hbm.at[:, pl.ds(0, _CH * 128)], buf, sem).wait()

            nit = (nch + 2) // 3

            def it_body(it, carry):
                for q in range(3):
                    cc = it * 3 + q
                    buf, sem = bufs[q]
                    nbuf, nsem = bufs[(q + 2) % 3]

                    @pl.when(cc < nch)
                    def _():
                        waitc(buf, sem)

                        @pl.when(cc + 2 < nch)
                        def _():
                            fire(cc + 2, nbuf, nsem)

                        process(cc, buf)

                return carry

            lax.fori_loop(0, nit, it_body, 0)

        phase(uid_hbm, utt_hbm, uscr_hbm)
        phase(mid_hbm, mtt_hbm, mscr_hbm)

        rem = jnp.minimum(iss_s[0], _NST)

        def drain(i, carry):
            pltpu.make_async_copy(uscr_hbm.at[pl.ds(0, D)],
                                  stage_v.at[pl.ds(0, D)], sem_w).wait()
            return carry

        lax.fori_loop(0, rem, drain, 0)

    return k


@functools.lru_cache(maxsize=None)
def _make_dot(B, D):
    b_per_w = B // _NW
    n_groups = b_per_w // _L
    mesh = plsc.VectorSubcoreMesh(
        core_axis_name="c", subcore_axis_name="s",
        num_cores=_NC, num_subcores=_NS,
    )

    @functools.partial(
        pl.kernel,
        out_type=jax.ShapeDtypeStruct((B,), jnp.float32),
        mesh=mesh,
        scratch_types=[
            pltpu.VMEM((b_per_w * D,), jnp.float32),
            pltpu.VMEM((b_per_w * D,), jnp.float32),
            pltpu.VMEM((b_per_w,), jnp.float32),
            pltpu.SemaphoreType.DMA,
            pltpu.SemaphoreType.DMA,
        ],
        compiler_params=pltpu.CompilerParams(needs_layout_passes=False),
    )
    def k(uscr_hbm, mscr_hbm, out_hbm, ubuf_v, mbuf_v, res_v, sem_u, sem_m):
        wid = lax.axis_index("s") * _NC + lax.axis_index("c")
        base = wid * b_per_w
        cu = pltpu.async_copy(uscr_hbm.at[pl.ds(base * D, b_per_w * D)],
                              ubuf_v, sem_u)
        cm = pltpu.async_copy(mscr_hbm.at[pl.ds(base * D, b_per_w * D)],
                              mbuf_v, sem_m)
        cu.wait()
        cm.wait()
        lane = lax.broadcasted_iota(jnp.int32, (_L,), 0)

        def body(g, carry):
            acc = jnp.zeros((_L,), jnp.float32)
            for j in range(_L):
                r = (g * _L + j) * D
                p = (ubuf_v[pl.ds(r, _L)] * mbuf_v[pl.ds(r, _L)])
                for c in range(1, D // _L):
                    p = p + (ubuf_v[pl.ds(r + c * _L, _L)]
                             * mbuf_v[pl.ds(r + c * _L, _L)])
                s = jnp.sum(p)
                acc = jnp.where(lane == j, s, acc)
            res_v[pl.ds(g * _L, _L)] = acc
            return carry

        lax.fori_loop(0, n_groups, body, 0)
        pltpu.sync_copy(res_v, out_hbm.at[pl.ds(base, b_per_w)])

    return k


def kernel(user_id, movie_id, user_table, movie_table):
    B = user_id.shape[0]
    V, D = user_table.shape
    uscr, mscr = _make_extract(B, D, V)(user_id, movie_id,
                                        user_table.T, movie_table.T)
    out = _make_dot(B, D)(uscr, mscr)
    return out.reshape(B, 1)
